# baseline (speedup 1.0000x reference)
import functools

import jax
import jax.numpy as jnp
from jax.experimental import pallas as pl
from jax.experimental.pallas import tpu as pltpu

S = 2048
D = 1024
N_QK = 2048
N_V = 2048
N_KNOW = 4096
D_SPACE = 64
N_HEADS = 16
D_HEAD = D // N_HEADS
MAX_K_QK = 32
MAX_K_V = 32
MAX_K_KNOW = 64

F32_INF_BITS = 0x7F800000


def _layer_norm(x, scale, bias, eps=1e-06):
    mean = jnp.mean(x, axis=-1, keepdims=True)
    var = jnp.mean(jnp.square(x - mean), axis=-1, keepdims=True)
    return (x - mean) / jnp.sqrt(var + eps) * scale + bias


def _gate_dense(scores, tau, k):
    raw = scores - tau
    gate = jnp.where(raw > 0, raw, 1e-08 * jnp.exp(raw))
    eg = jnp.exp(gate) - 1.0
    rows, n = eg.shape
    bits = jax.lax.bitcast_convert_type(eg, jnp.int32)

    def body(_, lohi):
        lo, hi = lohi
        mid = lo + ((hi - lo) >> 1)
        cnt = jnp.sum((bits >= mid).astype(jnp.int32), axis=1, keepdims=True)
        ge = cnt >= k
        return jnp.where(ge, mid, lo), jnp.where(ge, hi, mid)

    lo0 = jnp.zeros((rows, 1), jnp.int32)
    hi0 = jnp.full((rows, 1), F32_INF_BITS, jnp.int32)
    lo, hi = jax.lax.fori_loop(0, 15, body, (lo0, hi0))

    cnt_hi = jnp.sum((bits >= hi).astype(jnp.int32), axis=1, keepdims=True)
    j = k - cnt_hi
    band = (bits >= lo) & (bits < hi)
    t_f = jax.lax.bitcast_convert_type(lo, jnp.float32)
    thr_v = t_f
    mprev = jnp.full((rows, 1), jnp.inf, jnp.float32)
    for i in range(1, 4):
        cand = jnp.where(band & (eg < mprev), eg, -1.0)
        mprev = jnp.max(cand, axis=1, keepdims=True)
        thr_v = jnp.where(j == i, mprev, thr_v)
    thr_v = jnp.where(j > 3, mprev, thr_v)

    eg_m = jnp.where(eg >= thr_v, eg, 0.0)
    gsum = jnp.sum(eg_m, axis=1, keepdims=True) + 1e-08
    strength = jnp.tanh(jnp.max(eg_m, axis=1, keepdims=True))
    return eg_m * (strength / gsum)


def _norm_kernel(ne_ref, out_ref):
    ne = ne_ref[...]
    nrm = jnp.sqrt(jnp.sum(ne * ne, axis=1, keepdims=True))
    out_ref[...] = ne / (nrm + 1e-08)


def _route_attn_kernel(x_ref, s1_ref, b1_ref, wc_ref, bc_ref, qkl_ref, vl_ref,
                       h_ref, gq_ref, gk_ref, gv_ref, cs_ref):
    x = x_ref[...]
    h = _layer_norm(x, s1_ref[...], b1_ref[...])
    hp = jnp.dot(h, wc_ref[...], preferred_element_type=jnp.float32) + bc_ref[...]
    h_q = hp[:, 0:D_SPACE]
    h_k = hp[:, D_SPACE:2 * D_SPACE]
    h_v = hp[:, 2 * D_SPACE:3 * D_SPACE]
    tau = hp[:, 3 * D_SPACE:3 * D_SPACE + 3]
    sc_q = jnp.dot(h_q, qkl_ref[...], preferred_element_type=jnp.float32)
    sc_k = jnp.dot(h_k, qkl_ref[...], preferred_element_type=jnp.float32)
    sc_v = jnp.dot(h_v, vl_ref[...], preferred_element_type=jnp.float32)
    g_q = _gate_dense(sc_q, tau[:, 0:1], MAX_K_QK)
    g_k = _gate_dense(sc_k, tau[:, 1:2], MAX_K_QK)
    g_v = _gate_dense(sc_v, tau[:, 2:3], MAX_K_V)
    h_ref[...] = h.astype(jnp.bfloat16)
    gq_ref[...] = g_q.astype(jnp.bfloat16)
    gk_ref[...] = g_k.astype(jnp.bfloat16)
    gv_ref[...] = g_v.astype(jnp.bfloat16)
    cs_ref[0, 0, :] = jnp.sum(g_q, axis=0)
    cs_ref[0, 1, :] = jnp.sum(g_k, axis=0)
    cs_ref[0, 2, :] = jnp.sum(g_v, axis=0)


def _attn_kernel(q_ref, k_ref, v_ref, o_ref, s0_ref, s1_ref, p0_ref, p1_ref,
                 *, blk_q, blk_k):
    qi = pl.program_id(1)
    q = q_ref[...]
    scale = 1.0 / jnp.sqrt(jnp.float32(D_HEAD))
    dr = jax.lax.broadcasted_iota(jnp.int32, (blk_q, blk_k), 0)
    dc = jax.lax.broadcasted_iota(jnp.int32, (blk_q, blk_k), 1)
    diag_ok = dc <= dr

    s0_ref[...] = jnp.full((blk_q, S), -1e30, jnp.float32)
    s1_ref[...] = jnp.full((blk_q, S), -1e30, jnp.float32)

    def score_chunk(j, _):
        kc = k_ref[pl.ds(j * blk_k, blk_k), :]
        keep = jnp.logical_or(j != qi, diag_ok)
        c0 = jax.lax.dot_general(q[:, :D_HEAD], kc[:, :D_HEAD],
                                 (((1,), (1,)), ((), ())),
                                 preferred_element_type=jnp.float32) * scale
        c1 = jax.lax.dot_general(q[:, D_HEAD:], kc[:, D_HEAD:],
                                 (((1,), (1,)), ((), ())),
                                 preferred_element_type=jnp.float32) * scale
        s0_ref[:, pl.ds(j * blk_k, blk_k)] = jnp.where(keep, c0, -1e30)
        s1_ref[:, pl.ds(j * blk_k, blk_k)] = jnp.where(keep, c1, -1e30)
        return 0

    jax.lax.fori_loop(0, qi + 1, score_chunk, 0)

    s0 = s0_ref[...]
    m0 = jnp.max(s0, axis=1, keepdims=True)
    e0 = jnp.exp(s0 - m0)
    l0 = jnp.sum(e0, axis=1, keepdims=True)
    p0_ref[...] = e0.astype(jnp.bfloat16)
    s1 = s1_ref[...]
    m1 = jnp.max(s1, axis=1, keepdims=True)
    e1 = jnp.exp(s1 - m1)
    l1 = jnp.sum(e1, axis=1, keepdims=True)
    p1_ref[...] = e1.astype(jnp.bfloat16)

    def av_chunk(j, carry):
        o0, o1 = carry
        vc = v_ref[pl.ds(j * blk_k, blk_k), :]
        o0 += jnp.dot(p0_ref[:, pl.ds(j * blk_k, blk_k)], vc[:, :D_HEAD],
                      preferred_element_type=jnp.float32)
        o1 += jnp.dot(p1_ref[:, pl.ds(j * blk_k, blk_k)], vc[:, D_HEAD:],
                      preferred_element_type=jnp.float32)
        return o0, o1

    o0 = jnp.zeros((blk_q, D_HEAD), jnp.float32)
    o1 = jnp.zeros((blk_q, D_HEAD), jnp.float32)
    o0, o1 = jax.lax.fori_loop(0, qi + 1, av_chunk, (o0, o1))
    o_ref[...] = jnp.concatenate([o0 / l0, o1 / l1], axis=1)


def _route_know_kernel(x_ref, a_ref, wo_ref, s2_ref, b2_ref, wc_ref, bc_ref,
                       kl_ref, x1_ref, h2_ref, gk_ref, cs_ref):
    x1 = x_ref[...] + jnp.dot(a_ref[...].astype(jnp.bfloat16), wo_ref[...],
                              preferred_element_type=jnp.float32)
    h2 = _layer_norm(x1, s2_ref[...], b2_ref[...])
    hp = jnp.dot(h2, wc_ref[...], preferred_element_type=jnp.float32) + bc_ref[...]
    hk = hp[:, 0:D_SPACE]
    tau = hp[:, D_SPACE:D_SPACE + 1]
    sc = jnp.dot(hk, kl_ref[...], preferred_element_type=jnp.float32)
    g = _gate_dense(sc, tau, MAX_K_KNOW)
    x1_ref[...] = x1
    h2_ref[...] = h2.astype(jnp.bfloat16)
    gk_ref[...] = g.astype(jnp.bfloat16)
    cs_ref[0, 0, :] = jnp.sum(g, axis=0)


def _emit_qkv_kernel(h_ref, qe_ref, qw_ref, ve_ref, vw_ref,
                     gq_ref, gk_ref, gv_ref, q_ref, k_ref, v_ref,
                     acc_q, acc_k, acc_v, *, n_nt):
    n = pl.program_id(1)
    h = h_ref[...]
    qw = qw_ref[...]
    a_qk = jax.lax.dot_general(h, qe_ref[...], (((1,), (1,)), ((), ())),
                               preferred_element_type=jnp.float32)
    a_v = jax.lax.dot_general(h, ve_ref[...], (((1,), (1,)), ((), ())),
                              preferred_element_type=jnp.float32)
    g_q = gq_ref[...].astype(jnp.float32)
    g_k = gk_ref[...].astype(jnp.float32)
    g_v = gv_ref[...].astype(jnp.float32)
    qc = jnp.dot((a_qk * g_q).astype(jnp.bfloat16), qw,
                 preferred_element_type=jnp.float32)
    kc = jnp.dot((a_qk * g_k).astype(jnp.bfloat16), qw,
                 preferred_element_type=jnp.float32)
    vc = jnp.dot((a_v * g_v).astype(jnp.bfloat16), vw_ref[...],
                 preferred_element_type=jnp.float32)

    @pl.when(n == 0)
    def _():
        acc_q[...] = qc
        acc_k[...] = kc
        acc_v[...] = vc

    @pl.when(n > 0)
    def _():
        acc_q[...] += qc
        acc_k[...] += kc
        acc_v[...] += vc

    @pl.when(n == n_nt - 1)
    def _():
        q_ref[...] = acc_q[...].astype(jnp.bfloat16)
        k_ref[...] = acc_k[...].astype(jnp.bfloat16)
        v_ref[...] = acc_v[...].astype(jnp.bfloat16)


def _emit_know_kernel(h2_ref, x1_ref, ke_ref, kw_ref, g_ref, o_ref):
    n = pl.program_id(1)
    h2 = h2_ref[...]
    a = jax.lax.dot_general(h2, ke_ref[...], (((1,), (1,)), ((), ())),
                            preferred_element_type=jnp.float32)
    oc = jnp.dot((a * g_ref[...].astype(jnp.float32)).astype(jnp.bfloat16),
                 kw_ref[...], preferred_element_type=jnp.float32)

    @pl.when(n == 0)
    def _():
        o_ref[...] = x1_ref[...] + oc

    @pl.when(n > 0)
    def _():
        o_ref[...] += oc


def _aux_kernel(csa_ref, csk_ref, out_ref):
    csa = jnp.sum(csa_ref[...], axis=0) * (1.0 / S)
    csk = jnp.sum(csk_ref[...], axis=0) * (1.0 / S)
    t_qk = 1.0 / N_QK
    t_k = 1.0 / N_KNOW
    aux_attn = jnp.sum(jnp.square(csa - t_qk)) * N_QK
    aux_know = jnp.sum(jnp.square(csk - t_k)) * N_KNOW
    out_ref[...] = jnp.broadcast_to(aux_attn + aux_know, (1, 1))


def kernel(x, qk_emb, qk_w, v_emb, v_w, know_emb, know_w, neuron_emb,
           proj_attn_kernel, proj_attn_bias, tau_attn_kernel, tau_attn_bias,
           proj_know_kernel, proj_know_bias, tau_know_kernel, tau_know_bias,
           expand_O_kernel, ln1_scale, ln1_bias, ln2_scale, ln2_bias):
    f32 = jnp.float32
    x2 = x.reshape(S, D)

    emb_norm = pl.pallas_call(
        _norm_kernel,
        out_shape=jax.ShapeDtypeStruct((N_QK + N_V + N_KNOW, D_SPACE), f32),
    )(neuron_emb)
    qk_low_t = emb_norm[:N_QK].T
    v_low_t = emb_norm[N_QK:N_QK + N_V].T
    know_low_t = emb_norm[N_QK + N_V:].T

    wc_attn = jnp.concatenate([proj_attn_kernel, tau_attn_kernel], axis=1)
    wc_attn = jnp.pad(wc_attn, ((0, 0), (0, 256 - 3 * D_SPACE - 3)))
    bc_attn = jnp.concatenate([proj_attn_bias, tau_attn_bias])
    bc_attn = jnp.pad(bc_attn, (0, 256 - 3 * D_SPACE - 3)).reshape(1, 256)
    wc_know = jnp.concatenate([proj_know_kernel, tau_know_kernel], axis=1)
    wc_know = jnp.pad(wc_know, ((0, 0), (0, 128 - D_SPACE - 1)))
    bc_know = jnp.concatenate([proj_know_bias, tau_know_bias])
    bc_know = jnp.pad(bc_know, (0, 128 - D_SPACE - 1)).reshape(1, 128)

    ln1s = ln1_scale.reshape(1, D)
    ln1b = ln1_bias.reshape(1, D)
    ln2s = ln2_scale.reshape(1, D)
    ln2b = ln2_bias.reshape(1, D)

    blk = 256
    n_s = S // blk
    full = lambda shape: pl.BlockSpec(shape, lambda i: (0,) * len(shape))
    h, g_q, g_k, g_v, cs_attn = pl.pallas_call(
        _route_attn_kernel,
        grid=(n_s,),
        in_specs=[
            pl.BlockSpec((blk, D), lambda i: (i, 0)),
            full((1, D)), full((1, D)), full((D, 256)), full((1, 256)),
            full((D_SPACE, N_QK)), full((D_SPACE, N_V)),
        ],
        out_specs=[
            pl.BlockSpec((blk, D), lambda i: (i, 0)),
            pl.BlockSpec((blk, N_QK), lambda i: (i, 0)),
            pl.BlockSpec((blk, N_QK), lambda i: (i, 0)),
            pl.BlockSpec((blk, N_V), lambda i: (i, 0)),
            pl.BlockSpec((1, 3, N_QK), lambda i: (i, 0, 0)),
        ],
        out_shape=[
            jax.ShapeDtypeStruct((S, D), jnp.bfloat16),
            jax.ShapeDtypeStruct((S, N_QK), jnp.bfloat16),
            jax.ShapeDtypeStruct((S, N_QK), jnp.bfloat16),
            jax.ShapeDtypeStruct((S, N_V), jnp.bfloat16),
            jax.ShapeDtypeStruct((n_s, 3, N_QK), f32),
        ],
    )(x2, ln1s, ln1b, wc_attn, bc_attn, qk_low_t, v_low_t)

    blk_s, blk_n = 512, 512
    n_st, n_nt = S // blk_s, N_QK // blk_n
    q, kk, vv = pl.pallas_call(
        functools.partial(_emit_qkv_kernel, n_nt=N_QK // blk_n),
        grid=(n_st, n_nt),
        in_specs=[
            pl.BlockSpec((blk_s, D), lambda i, j: (i, 0)),
            pl.BlockSpec((blk_n, D), lambda i, j: (j, 0)),
            pl.BlockSpec((blk_n, D), lambda i, j: (j, 0)),
            pl.BlockSpec((blk_n, D), lambda i, j: (j, 0)),
            pl.BlockSpec((blk_n, D), lambda i, j: (j, 0)),
            pl.BlockSpec((blk_s, blk_n), lambda i, j: (i, j)),
            pl.BlockSpec((blk_s, blk_n), lambda i, j: (i, j)),
            pl.BlockSpec((blk_s, blk_n), lambda i, j: (i, j)),
        ],
        out_specs=[
            pl.BlockSpec((blk_s, D), lambda i, j: (i, 0)),
            pl.BlockSpec((blk_s, D), lambda i, j: (i, 0)),
            pl.BlockSpec((blk_s, D), lambda i, j: (i, 0)),
        ],
        out_shape=[jax.ShapeDtypeStruct((S, D), jnp.bfloat16)] * 3,
        scratch_shapes=[pltpu.VMEM((blk_s, D), f32)] * 3,
    )(h, qk_emb.astype(jnp.bfloat16), qk_w.astype(jnp.bfloat16),
      v_emb.astype(jnp.bfloat16), v_w.astype(jnp.bfloat16), g_q, g_k, g_v)

    blk_q = 512
    attn_flat = pl.pallas_call(
        functools.partial(_attn_kernel, blk_q=blk_q, blk_k=blk_q),
        grid=(N_HEADS // 2, S // blk_q),
        in_specs=[
            pl.BlockSpec((blk_q, 2 * D_HEAD), lambda hh, i: (i, hh)),
            pl.BlockSpec((S, 2 * D_HEAD), lambda hh, i: (0, hh)),
            pl.BlockSpec((S, 2 * D_HEAD), lambda hh, i: (0, hh)),
        ],
        out_specs=pl.BlockSpec((blk_q, 2 * D_HEAD), lambda hh, i: (i, hh)),
        out_shape=jax.ShapeDtypeStruct((S, D), f32),
        scratch_shapes=[pltpu.VMEM((blk_q, S), f32),
                        pltpu.VMEM((blk_q, S), f32),
                        pltpu.VMEM((blk_q, S), jnp.bfloat16),
                        pltpu.VMEM((blk_q, S), jnp.bfloat16)],
    )(q, kk, vv)

    x1, h2, g_kn, cs_know = pl.pallas_call(
        _route_know_kernel,
        grid=(n_s,),
        in_specs=[
            pl.BlockSpec((blk, D), lambda i: (i, 0)),
            pl.BlockSpec((blk, D), lambda i: (i, 0)),
            full((D, D)), full((1, D)), full((1, D)),
            full((D, 128)), full((1, 128)), full((D_SPACE, N_KNOW)),
        ],
        out_specs=[
            pl.BlockSpec((blk, D), lambda i: (i, 0)),
            pl.BlockSpec((blk, D), lambda i: (i, 0)),
            pl.BlockSpec((blk, N_KNOW), lambda i: (i, 0)),
            pl.BlockSpec((1, 1, N_KNOW), lambda i: (i, 0, 0)),
        ],
        out_shape=[
            jax.ShapeDtypeStruct((S, D), f32),
            jax.ShapeDtypeStruct((S, D), jnp.bfloat16),
            jax.ShapeDtypeStruct((S, N_KNOW), jnp.bfloat16),
            jax.ShapeDtypeStruct((n_s, 1, N_KNOW), f32),
        ],
    )(x2, attn_flat, expand_O_kernel.astype(jnp.bfloat16), ln2s, ln2b,
      wc_know, bc_know, know_low_t)

    n_nt_k = N_KNOW // blk_n
    out = pl.pallas_call(
        _emit_know_kernel,
        grid=(n_st, n_nt_k),
        in_specs=[
            pl.BlockSpec((blk_s, D), lambda i, j: (i, 0)),
            pl.BlockSpec((blk_s, D), lambda i, j: (i, 0)),
            pl.BlockSpec((blk_n, D), lambda i, j: (j, 0)),
            pl.BlockSpec((blk_n, D), lambda i, j: (j, 0)),
            pl.BlockSpec((blk_s, blk_n), lambda i, j: (i, j)),
        ],
        out_specs=pl.BlockSpec((blk_s, D), lambda i, j: (i, 0)),
        out_shape=jax.ShapeDtypeStruct((S, D), f32),
    )(h2, x1, know_emb.astype(jnp.bfloat16), know_w.astype(jnp.bfloat16), g_kn)

    aux = pl.pallas_call(
        _aux_kernel,
        in_specs=[
            pl.BlockSpec((n_s, 3, N_QK), lambda: (0, 0, 0)),
            pl.BlockSpec((n_s, 1, N_KNOW), lambda: (0, 0, 0)),
        ],
        out_specs=pl.BlockSpec((1, 1), lambda: (0, 0)),
        out_shape=jax.ShapeDtypeStruct((1, 1), f32),
    )(cs_attn, cs_know)

    return out.reshape(1, S, D), aux[0, 0]


# speedup vs baseline: 1.0409x; 1.0409x over previous
import functools

import jax
import jax.numpy as jnp
from jax.experimental import pallas as pl
from jax.experimental.pallas import tpu as pltpu

S = 2048
D = 1024
N_QK = 2048
N_V = 2048
N_KNOW = 4096
D_SPACE = 64
N_HEADS = 16
D_HEAD = D // N_HEADS
MAX_K_QK = 32
MAX_K_V = 32
MAX_K_KNOW = 64

F32_INF_BITS = 0x7F800000


def _layer_norm(x, scale, bias, eps=1e-06):
    mean = jnp.mean(x, axis=-1, keepdims=True)
    var = jnp.mean(jnp.square(x - mean), axis=-1, keepdims=True)
    return (x - mean) / jnp.sqrt(var + eps) * scale + bias


def _gate_dense(scores, tau, k):
    raw = scores - tau
    gate = jnp.where(raw > 0, raw, 1e-08 * jnp.exp(raw))
    eg = jnp.exp(gate) - 1.0
    rows, n = eg.shape
    bits = jax.lax.bitcast_convert_type(eg, jnp.int32)

    def body(_, lohi):
        lo, hi = lohi
        mid = lo + ((hi - lo) >> 1)
        cnt = jnp.sum((bits >= mid).astype(jnp.int32), axis=1, keepdims=True)
        ge = cnt >= k
        return jnp.where(ge, mid, lo), jnp.where(ge, hi, mid)

    lo0 = jnp.zeros((rows, 1), jnp.int32)
    hi0 = jnp.full((rows, 1), F32_INF_BITS, jnp.int32)
    lo, hi = jax.lax.fori_loop(0, 15, body, (lo0, hi0))

    cnt_hi = jnp.sum((bits >= hi).astype(jnp.int32), axis=1, keepdims=True)
    j = k - cnt_hi
    band = (bits >= lo) & (bits < hi)
    t_f = jax.lax.bitcast_convert_type(lo, jnp.float32)
    thr_v = t_f
    mprev = jnp.full((rows, 1), jnp.inf, jnp.float32)
    for i in range(1, 4):
        cand = jnp.where(band & (eg < mprev), eg, -1.0)
        mprev = jnp.max(cand, axis=1, keepdims=True)
        thr_v = jnp.where(j == i, mprev, thr_v)
    thr_v = jnp.where(j > 3, mprev, thr_v)

    eg_m = jnp.where(eg >= thr_v, eg, 0.0)
    gsum = jnp.sum(eg_m, axis=1, keepdims=True) + 1e-08
    strength = jnp.tanh(jnp.max(eg_m, axis=1, keepdims=True))
    return eg_m * (strength / gsum)


def _norm_kernel(ne_ref, out_ref):
    ne = ne_ref[...]
    nrm = jnp.sqrt(jnp.sum(ne * ne, axis=1, keepdims=True))
    out_ref[...] = ne / (nrm + 1e-08)


def _route_attn_kernel(x_ref, s1_ref, b1_ref, wc_ref, bc_ref, qkl_ref, vl_ref,
                       h_ref, gq_ref, gk_ref, gv_ref, cs_ref):
    x = x_ref[...]
    h = _layer_norm(x, s1_ref[...], b1_ref[...])
    hp = jnp.dot(h, wc_ref[...], preferred_element_type=jnp.float32) + bc_ref[...]
    h_q = hp[:, 0:D_SPACE]
    h_k = hp[:, D_SPACE:2 * D_SPACE]
    h_v = hp[:, 2 * D_SPACE:3 * D_SPACE]
    tau = hp[:, 3 * D_SPACE:3 * D_SPACE + 3]
    sc_q = jnp.dot(h_q, qkl_ref[...], preferred_element_type=jnp.float32)
    sc_k = jnp.dot(h_k, qkl_ref[...], preferred_element_type=jnp.float32)
    sc_v = jnp.dot(h_v, vl_ref[...], preferred_element_type=jnp.float32)
    g_q = _gate_dense(sc_q, tau[:, 0:1], MAX_K_QK)
    g_k = _gate_dense(sc_k, tau[:, 1:2], MAX_K_QK)
    g_v = _gate_dense(sc_v, tau[:, 2:3], MAX_K_V)
    h_ref[...] = h.astype(jnp.bfloat16)
    gq_ref[...] = g_q.astype(jnp.bfloat16)
    gk_ref[...] = g_k.astype(jnp.bfloat16)
    gv_ref[...] = g_v.astype(jnp.bfloat16)
    cs_ref[0, 0, :] = jnp.sum(g_q, axis=0)
    cs_ref[0, 1, :] = jnp.sum(g_k, axis=0)
    cs_ref[0, 2, :] = jnp.sum(g_v, axis=0)


def _attn_kernel(q_ref, k_ref, v_ref, o_ref, *, blk_q, blk_k):
    qi = pl.program_id(1)
    q = q_ref[...]
    scale = 1.0 / jnp.sqrt(jnp.float32(D_HEAD))

    def one_head(q1, kc, vc, carry, rows, cols):
        o, m, l = carry
        s = jax.lax.dot_general(q1, kc, (((1,), (1,)), ((), ())),
                                preferred_element_type=jnp.float32) * scale
        s = jnp.where(cols <= rows, s, -1e30)
        m_new = jnp.maximum(m, jnp.max(s, axis=1, keepdims=True))
        alpha = jnp.exp(m - m_new)
        p = jnp.exp(s - m_new)
        l = l * alpha + jnp.sum(p, axis=1, keepdims=True)
        o = o * alpha + jnp.dot(p.astype(jnp.bfloat16), vc,
                                preferred_element_type=jnp.float32)
        return o, m_new, l

    def body(j, carry):
        c0, c1 = carry
        kc = k_ref[pl.ds(j * blk_k, blk_k), :]
        vc = v_ref[pl.ds(j * blk_k, blk_k), :]
        rows = (jax.lax.broadcasted_iota(jnp.int32, (blk_q, blk_k), 0)
                + qi * blk_q)
        cols = (jax.lax.broadcasted_iota(jnp.int32, (blk_q, blk_k), 1)
                + j * blk_k)
        c0 = one_head(q[:, :D_HEAD], kc[:, :D_HEAD], vc[:, :D_HEAD],
                      c0, rows, cols)
        c1 = one_head(q[:, D_HEAD:], kc[:, D_HEAD:], vc[:, D_HEAD:],
                      c1, rows, cols)
        return c0, c1

    def init():
        return (jnp.zeros((blk_q, D_HEAD), jnp.float32),
                jnp.full((blk_q, 1), -1e30, jnp.float32),
                jnp.zeros((blk_q, 1), jnp.float32))

    (o0, _, l0), (o1, _, l1) = jax.lax.fori_loop(0, qi + 1, body,
                                                 (init(), init()))
    o_ref[...] = jnp.concatenate([o0 / l0, o1 / l1], axis=1)


def _route_know_kernel(x_ref, a_ref, wo_ref, s2_ref, b2_ref, wc_ref, bc_ref,
                       kl_ref, x1_ref, h2_ref, gk_ref, cs_ref):
    x1 = x_ref[...] + jnp.dot(a_ref[...].astype(jnp.bfloat16), wo_ref[...],
                              preferred_element_type=jnp.float32)
    h2 = _layer_norm(x1, s2_ref[...], b2_ref[...])
    hp = jnp.dot(h2, wc_ref[...], preferred_element_type=jnp.float32) + bc_ref[...]
    hk = hp[:, 0:D_SPACE]
    tau = hp[:, D_SPACE:D_SPACE + 1]
    sc = jnp.dot(hk, kl_ref[...], preferred_element_type=jnp.float32)
    g = _gate_dense(sc, tau, MAX_K_KNOW)
    x1_ref[...] = x1
    h2_ref[...] = h2.astype(jnp.bfloat16)
    gk_ref[...] = g.astype(jnp.bfloat16)
    cs_ref[0, 0, :] = jnp.sum(g, axis=0)


def _emit_qkv_kernel(h_ref, qe_ref, qw_ref, ve_ref, vw_ref,
                     gq_ref, gk_ref, gv_ref, q_ref, k_ref, v_ref,
                     acc_q, acc_k, acc_v, *, n_nt):
    n = pl.program_id(1)
    h = h_ref[...]
    qw = qw_ref[...]
    a_qk = jax.lax.dot_general(h, qe_ref[...], (((1,), (1,)), ((), ())),
                               preferred_element_type=jnp.float32)
    a_v = jax.lax.dot_general(h, ve_ref[...], (((1,), (1,)), ((), ())),
                              preferred_element_type=jnp.float32)
    g_q = gq_ref[...].astype(jnp.float32)
    g_k = gk_ref[...].astype(jnp.float32)
    g_v = gv_ref[...].astype(jnp.float32)
    qc = jnp.dot((a_qk * g_q).astype(jnp.bfloat16), qw,
                 preferred_element_type=jnp.float32)
    kc = jnp.dot((a_qk * g_k).astype(jnp.bfloat16), qw,
                 preferred_element_type=jnp.float32)
    vc = jnp.dot((a_v * g_v).astype(jnp.bfloat16), vw_ref[...],
                 preferred_element_type=jnp.float32)

    @pl.when(n == 0)
    def _():
        acc_q[...] = qc
        acc_k[...] = kc
        acc_v[...] = vc

    @pl.when(n > 0)
    def _():
        acc_q[...] += qc
        acc_k[...] += kc
        acc_v[...] += vc

    @pl.when(n == n_nt - 1)
    def _():
        q_ref[...] = acc_q[...].astype(jnp.bfloat16)
        k_ref[...] = acc_k[...].astype(jnp.bfloat16)
        v_ref[...] = acc_v[...].astype(jnp.bfloat16)


def _emit_know_kernel(h2_ref, x1_ref, ke_ref, kw_ref, g_ref, o_ref):
    n = pl.program_id(1)
    h2 = h2_ref[...]
    a = jax.lax.dot_general(h2, ke_ref[...], (((1,), (1,)), ((), ())),
                            preferred_element_type=jnp.float32)
    oc = jnp.dot((a * g_ref[...].astype(jnp.float32)).astype(jnp.bfloat16),
                 kw_ref[...], preferred_element_type=jnp.float32)

    @pl.when(n == 0)
    def _():
        o_ref[...] = x1_ref[...] + oc

    @pl.when(n > 0)
    def _():
        o_ref[...] += oc


def _aux_kernel(csa_ref, csk_ref, out_ref):
    csa = jnp.sum(csa_ref[...], axis=0) * (1.0 / S)
    csk = jnp.sum(csk_ref[...], axis=0) * (1.0 / S)
    t_qk = 1.0 / N_QK
    t_k = 1.0 / N_KNOW
    aux_attn = jnp.sum(jnp.square(csa - t_qk)) * N_QK
    aux_know = jnp.sum(jnp.square(csk - t_k)) * N_KNOW
    out_ref[...] = jnp.broadcast_to(aux_attn + aux_know, (1, 1))


def kernel(x, qk_emb, qk_w, v_emb, v_w, know_emb, know_w, neuron_emb,
           proj_attn_kernel, proj_attn_bias, tau_attn_kernel, tau_attn_bias,
           proj_know_kernel, proj_know_bias, tau_know_kernel, tau_know_bias,
           expand_O_kernel, ln1_scale, ln1_bias, ln2_scale, ln2_bias):
    f32 = jnp.float32
    x2 = x.reshape(S, D)

    emb_norm = pl.pallas_call(
        _norm_kernel,
        out_shape=jax.ShapeDtypeStruct((N_QK + N_V + N_KNOW, D_SPACE), f32),
    )(neuron_emb)
    qk_low_t = emb_norm[:N_QK].T
    v_low_t = emb_norm[N_QK:N_QK + N_V].T
    know_low_t = emb_norm[N_QK + N_V:].T

    wc_attn = jnp.concatenate([proj_attn_kernel, tau_attn_kernel], axis=1)
    wc_attn = jnp.pad(wc_attn, ((0, 0), (0, 256 - 3 * D_SPACE - 3)))
    bc_attn = jnp.concatenate([proj_attn_bias, tau_attn_bias])
    bc_attn = jnp.pad(bc_attn, (0, 256 - 3 * D_SPACE - 3)).reshape(1, 256)
    wc_know = jnp.concatenate([proj_know_kernel, tau_know_kernel], axis=1)
    wc_know = jnp.pad(wc_know, ((0, 0), (0, 128 - D_SPACE - 1)))
    bc_know = jnp.concatenate([proj_know_bias, tau_know_bias])
    bc_know = jnp.pad(bc_know, (0, 128 - D_SPACE - 1)).reshape(1, 128)

    ln1s = ln1_scale.reshape(1, D)
    ln1b = ln1_bias.reshape(1, D)
    ln2s = ln2_scale.reshape(1, D)
    ln2b = ln2_bias.reshape(1, D)

    blk = 256
    n_s = S // blk
    full = lambda shape: pl.BlockSpec(shape, lambda i: (0,) * len(shape))
    h, g_q, g_k, g_v, cs_attn = pl.pallas_call(
        _route_attn_kernel,
        grid=(n_s,),
        in_specs=[
            pl.BlockSpec((blk, D), lambda i: (i, 0)),
            full((1, D)), full((1, D)), full((D, 256)), full((1, 256)),
            full((D_SPACE, N_QK)), full((D_SPACE, N_V)),
        ],
        out_specs=[
            pl.BlockSpec((blk, D), lambda i: (i, 0)),
            pl.BlockSpec((blk, N_QK), lambda i: (i, 0)),
            pl.BlockSpec((blk, N_QK), lambda i: (i, 0)),
            pl.BlockSpec((blk, N_V), lambda i: (i, 0)),
            pl.BlockSpec((1, 3, N_QK), lambda i: (i, 0, 0)),
        ],
        out_shape=[
            jax.ShapeDtypeStruct((S, D), jnp.bfloat16),
            jax.ShapeDtypeStruct((S, N_QK), jnp.bfloat16),
            jax.ShapeDtypeStruct((S, N_QK), jnp.bfloat16),
            jax.ShapeDtypeStruct((S, N_V), jnp.bfloat16),
            jax.ShapeDtypeStruct((n_s, 3, N_QK), f32),
        ],
    )(x2, ln1s, ln1b, wc_attn, bc_attn, qk_low_t, v_low_t)

    blk_s, blk_n = 512, 512
    n_st, n_nt = S // blk_s, N_QK // blk_n
    q, kk, vv = pl.pallas_call(
        functools.partial(_emit_qkv_kernel, n_nt=N_QK // blk_n),
        grid=(n_st, n_nt),
        in_specs=[
            pl.BlockSpec((blk_s, D), lambda i, j: (i, 0)),
            pl.BlockSpec((blk_n, D), lambda i, j: (j, 0)),
            pl.BlockSpec((blk_n, D), lambda i, j: (j, 0)),
            pl.BlockSpec((blk_n, D), lambda i, j: (j, 0)),
            pl.BlockSpec((blk_n, D), lambda i, j: (j, 0)),
            pl.BlockSpec((blk_s, blk_n), lambda i, j: (i, j)),
            pl.BlockSpec((blk_s, blk_n), lambda i, j: (i, j)),
            pl.BlockSpec((blk_s, blk_n), lambda i, j: (i, j)),
        ],
        out_specs=[
            pl.BlockSpec((blk_s, D), lambda i, j: (i, 0)),
            pl.BlockSpec((blk_s, D), lambda i, j: (i, 0)),
            pl.BlockSpec((blk_s, D), lambda i, j: (i, 0)),
        ],
        out_shape=[jax.ShapeDtypeStruct((S, D), jnp.bfloat16)] * 3,
        scratch_shapes=[pltpu.VMEM((blk_s, D), f32)] * 3,
    )(h, qk_emb.astype(jnp.bfloat16), qk_w.astype(jnp.bfloat16),
      v_emb.astype(jnp.bfloat16), v_w.astype(jnp.bfloat16), g_q, g_k, g_v)

    blk_q = 512
    attn_flat = pl.pallas_call(
        functools.partial(_attn_kernel, blk_q=blk_q, blk_k=blk_q),
        grid=(N_HEADS // 2, S // blk_q),
        in_specs=[
            pl.BlockSpec((blk_q, 2 * D_HEAD), lambda hh, i: (i, hh)),
            pl.BlockSpec((S, 2 * D_HEAD), lambda hh, i: (0, hh)),
            pl.BlockSpec((S, 2 * D_HEAD), lambda hh, i: (0, hh)),
        ],
        out_specs=pl.BlockSpec((blk_q, 2 * D_HEAD), lambda hh, i: (i, hh)),
        out_shape=jax.ShapeDtypeStruct((S, D), f32),
    )(q, kk, vv)

    x1, h2, g_kn, cs_know = pl.pallas_call(
        _route_know_kernel,
        grid=(n_s,),
        in_specs=[
            pl.BlockSpec((blk, D), lambda i: (i, 0)),
            pl.BlockSpec((blk, D), lambda i: (i, 0)),
            full((D, D)), full((1, D)), full((1, D)),
            full((D, 128)), full((1, 128)), full((D_SPACE, N_KNOW)),
        ],
        out_specs=[
            pl.BlockSpec((blk, D), lambda i: (i, 0)),
            pl.BlockSpec((blk, D), lambda i: (i, 0)),
            pl.BlockSpec((blk, N_KNOW), lambda i: (i, 0)),
            pl.BlockSpec((1, 1, N_KNOW), lambda i: (i, 0, 0)),
        ],
        out_shape=[
            jax.ShapeDtypeStruct((S, D), f32),
            jax.ShapeDtypeStruct((S, D), jnp.bfloat16),
            jax.ShapeDtypeStruct((S, N_KNOW), jnp.bfloat16),
            jax.ShapeDtypeStruct((n_s, 1, N_KNOW), f32),
        ],
    )(x2, attn_flat, expand_O_kernel.astype(jnp.bfloat16), ln2s, ln2b,
      wc_know, bc_know, know_low_t)

    n_nt_k = N_KNOW // blk_n
    out = pl.pallas_call(
        _emit_know_kernel,
        grid=(n_st, n_nt_k),
        in_specs=[
            pl.BlockSpec((blk_s, D), lambda i, j: (i, 0)),
            pl.BlockSpec((blk_s, D), lambda i, j: (i, 0)),
            pl.BlockSpec((blk_n, D), lambda i, j: (j, 0)),
            pl.BlockSpec((blk_n, D), lambda i, j: (j, 0)),
            pl.BlockSpec((blk_s, blk_n), lambda i, j: (i, j)),
        ],
        out_specs=pl.BlockSpec((blk_s, D), lambda i, j: (i, 0)),
        out_shape=jax.ShapeDtypeStruct((S, D), f32),
    )(h2, x1, know_emb.astype(jnp.bfloat16), know_w.astype(jnp.bfloat16), g_kn)

    aux = pl.pallas_call(
        _aux_kernel,
        in_specs=[
            pl.BlockSpec((n_s, 3, N_QK), lambda: (0, 0, 0)),
            pl.BlockSpec((n_s, 1, N_KNOW), lambda: (0, 0, 0)),
        ],
        out_specs=pl.BlockSpec((1, 1), lambda: (0, 0)),
        out_shape=jax.ShapeDtypeStruct((1, 1), f32),
    )(cs_attn, cs_know)

    return out.reshape(1, S, D), aux[0, 0]
